# baseline (device time: 179322 ns/iter reference)
import jax
import jax.numpy as jnp
from jax import lax
from jax.experimental import pallas as pl
from jax.experimental.pallas import tpu as pltpu

N_DEV = 4
NBLK = 512
MCHUNK = 256


def _gemm(x, w_mat):
    M, K = x.shape
    _, N = w_mat.shape

    def body(x_ref, w_ref, y_ref, amax_ref):
        n = pl.program_id(0)
        part = jnp.maximum(
            jnp.dot(x_ref[...], w_ref[...],
                    preferred_element_type=jnp.float32),
            0.0,
        )
        y_ref[...] = part
        m = jnp.full((8, 128), jnp.max(part), jnp.float32)

        @pl.when(n == 0)
        def _():
            amax_ref[...] = m

        @pl.when(n != 0)
        def _():
            amax_ref[...] = jnp.maximum(amax_ref[...], m)

    return pl.pallas_call(
        body,
        grid=(N // NBLK,),
        out_shape=(
            jax.ShapeDtypeStruct((M, N), jnp.float32),
            jax.ShapeDtypeStruct((8, 128), jnp.float32),
        ),
        in_specs=[
            pl.BlockSpec((M, K), lambda n: (0, 0)),
            pl.BlockSpec((K, NBLK), lambda n: (0, n)),
        ],
        out_specs=(
            pl.BlockSpec((M, NBLK), lambda n: (0, n)),
            pl.BlockSpec((8, 128), lambda n: (0, 0)),
        ),
        compiler_params=pltpu.CompilerParams(
            dimension_semantics=("arbitrary",),
            vmem_limit_bytes=128 * 1024 * 1024,
        ),
    )(x, w_mat)


def _a2a(y, amax_local):
    M, N = y.shape
    NB = N // N_DEV

    def body(y_ref, amax_in, out_ref, bounce, qsend, qrecv, amax_buf,
             copy_sems, dsend_sems, drecv_sems, asend_sems, arecv_sems):
        my = lax.axis_index("i")

        amax_buf[0] = amax_in[...]
        amax_rdmas = []
        for d in range(1, N_DEV):
            tgt = lax.rem(my + d, N_DEV)
            r = pltpu.make_async_remote_copy(
                src_ref=amax_buf.at[0],
                dst_ref=amax_buf.at[d],
                send_sem=asend_sems.at[d],
                recv_sem=arecv_sems.at[d],
                device_id=(tgt,),
                device_id_type=pl.DeviceIdType.MESH,
            )
            r.start()
            amax_rdmas.append(r)

        reads = []
        for d in range(N_DEV):
            tgt = lax.rem(my + d, N_DEV)
            cp = pltpu.make_async_copy(
                y_ref.at[:, pl.ds(tgt * NB, NB)],
                bounce.at[d],
                copy_sems.at[d],
            )
            cp.start()
            reads.append(cp)

        for r in amax_rdmas:
            r.wait_recv()
        amax = jnp.max(amax_buf[...])
        scale = amax / 127.0

        data_rdmas = []
        for d in range(1, N_DEV):
            tgt = lax.rem(my + d, N_DEV)
            reads[d].wait()
            for c in range(M // MCHUNK):
                rows = pl.ds(c * MCHUNK, MCHUNK)
                q = jnp.clip(jnp.round(bounce[d, rows, :] / scale),
                             -127.0, 127.0)
                qsend[d - 1, rows, :] = q.astype(jnp.int8)
            r = pltpu.make_async_remote_copy(
                src_ref=qsend.at[d - 1],
                dst_ref=qrecv.at[d - 1],
                send_sem=dsend_sems.at[d],
                recv_sem=drecv_sems.at[d],
                device_id=(tgt,),
                device_id_type=pl.DeviceIdType.MESH,
            )
            r.start()
            data_rdmas.append(r)

        reads[0].wait()
        for c in range(M // MCHUNK):
            rows = pl.ds(c * MCHUNK, MCHUNK)
            q = jnp.clip(jnp.round(bounce[0, rows, :] / scale),
                         -127.0, 127.0)
            bounce[0, rows, :] = q * scale
        writes = [pltpu.make_async_copy(
            bounce.at[0], out_ref.at[pl.ds(my * M, M), :], copy_sems.at[0])]
        writes[0].start()

        for d in range(1, N_DEV):
            data_rdmas[d - 1].wait_recv()
            src = lax.rem(my - d + N_DEV, N_DEV)
            for c in range(M // MCHUNK):
                rows = pl.ds(c * MCHUNK, MCHUNK)
                bounce[d, rows, :] = (
                    qrecv[d - 1, rows, :].astype(jnp.float32) * scale)
            cp = pltpu.make_async_copy(
                bounce.at[d], out_ref.at[pl.ds(src * M, M), :],
                copy_sems.at[d])
            cp.start()
            writes.append(cp)

        for cp in writes:
            cp.wait()
        for r in amax_rdmas + data_rdmas:
            r.wait_send()

    return pl.pallas_call(
        body,
        out_shape=jax.ShapeDtypeStruct((N_DEV * M, NB), jnp.float32),
        in_specs=[
            pl.BlockSpec(memory_space=pl.ANY),
            pl.BlockSpec((8, 128), memory_space=pltpu.VMEM),
        ],
        out_specs=pl.BlockSpec(memory_space=pl.ANY),
        scratch_shapes=[
            pltpu.VMEM((N_DEV, M, NB), jnp.float32),
            pltpu.VMEM((N_DEV - 1, M, NB), jnp.int8),
            pltpu.VMEM((N_DEV - 1, M, NB), jnp.int8),
            pltpu.VMEM((N_DEV, 8, 128), jnp.float32),
            pltpu.SemaphoreType.DMA((N_DEV,)),
            pltpu.SemaphoreType.DMA((N_DEV,)),
            pltpu.SemaphoreType.DMA((N_DEV,)),
            pltpu.SemaphoreType.DMA((N_DEV,)),
            pltpu.SemaphoreType.DMA((N_DEV,)),
        ],
        compiler_params=pltpu.CompilerParams(
            vmem_limit_bytes=128 * 1024 * 1024,
        ),
    )(y, amax_local)


def kernel(x, w_mat):
    y, amax_local = _gemm(x, w_mat)
    return _a2a(y, amax_local)


# device time: 178763 ns/iter; 1.0031x vs baseline; 1.0031x over previous
import jax
import jax.numpy as jnp
from jax import lax
from jax.experimental import pallas as pl
from jax.experimental.pallas import tpu as pltpu

N_DEV = 4
NBLK = 512
MCHUNK = 256


def _gemm(x, w_mat):
    M, K = x.shape
    _, N = w_mat.shape

    def body(x_ref, w_ref, y_ref, amax_ref):
        n = pl.program_id(0)
        part = jnp.maximum(
            jnp.dot(x_ref[...], w_ref[...],
                    preferred_element_type=jnp.float32),
            0.0,
        )
        y_ref[...] = part
        m = jnp.full((8, 128), jnp.max(part), jnp.float32)

        @pl.when(n == 0)
        def _():
            amax_ref[...] = m

        @pl.when(n != 0)
        def _():
            amax_ref[...] = jnp.maximum(amax_ref[...], m)

    blks = (N // NBLK) // N_DEV
    return pl.pallas_call(
        body,
        grid=(N // NBLK,),
        out_shape=(
            jax.ShapeDtypeStruct((N_DEV * M, N // N_DEV), jnp.float32),
            jax.ShapeDtypeStruct((8, 128), jnp.float32),
        ),
        in_specs=[
            pl.BlockSpec((M, K), lambda n: (0, 0)),
            pl.BlockSpec((K, NBLK), lambda n: (0, n)),
        ],
        out_specs=(
            pl.BlockSpec((M, NBLK), lambda n: (n // blks, n % blks)),
            pl.BlockSpec((8, 128), lambda n: (0, 0)),
        ),
        compiler_params=pltpu.CompilerParams(
            dimension_semantics=("arbitrary",),
            vmem_limit_bytes=128 * 1024 * 1024,
        ),
    )(x, w_mat)


def _a2a(y, amax_local):
    NB = y.shape[1]
    M = y.shape[0] // N_DEV

    def body(y_ref, amax_in, out_ref, bounce, qsend, qrecv, amax_buf,
             copy_sems, dsend_sems, drecv_sems, asend_sems, arecv_sems):
        my = lax.axis_index("i")

        amax_buf[0] = amax_in[...]
        amax_rdmas = []
        for d in range(1, N_DEV):
            tgt = lax.rem(my + d, N_DEV)
            r = pltpu.make_async_remote_copy(
                src_ref=amax_buf.at[0],
                dst_ref=amax_buf.at[d],
                send_sem=asend_sems.at[d],
                recv_sem=arecv_sems.at[d],
                device_id=(tgt,),
                device_id_type=pl.DeviceIdType.MESH,
            )
            r.start()
            amax_rdmas.append(r)

        reads = []
        for d in range(N_DEV):
            tgt = lax.rem(my + d, N_DEV)
            cp = pltpu.make_async_copy(
                y_ref.at[pl.ds(tgt * M, M), :],
                bounce.at[d],
                copy_sems.at[d],
            )
            cp.start()
            reads.append(cp)

        for r in amax_rdmas:
            r.wait_recv()
        amax = jnp.max(amax_buf[...])
        scale = amax / 127.0

        data_rdmas = []
        for d in range(1, N_DEV):
            tgt = lax.rem(my + d, N_DEV)
            reads[d].wait()
            for c in range(M // MCHUNK):
                rows = pl.ds(c * MCHUNK, MCHUNK)
                q = jnp.clip(jnp.round(bounce[d, rows, :] / scale),
                             -127.0, 127.0)
                qsend[d - 1, rows, :] = q.astype(jnp.int8)
            r = pltpu.make_async_remote_copy(
                src_ref=qsend.at[d - 1],
                dst_ref=qrecv.at[d - 1],
                send_sem=dsend_sems.at[d],
                recv_sem=drecv_sems.at[d],
                device_id=(tgt,),
                device_id_type=pl.DeviceIdType.MESH,
            )
            r.start()
            data_rdmas.append(r)

        reads[0].wait()
        for c in range(M // MCHUNK):
            rows = pl.ds(c * MCHUNK, MCHUNK)
            q = jnp.clip(jnp.round(bounce[0, rows, :] / scale),
                         -127.0, 127.0)
            bounce[0, rows, :] = q * scale
        writes = [pltpu.make_async_copy(
            bounce.at[0], out_ref.at[pl.ds(my * M, M), :], copy_sems.at[0])]
        writes[0].start()

        for d in range(1, N_DEV):
            data_rdmas[d - 1].wait_recv()
            src = lax.rem(my - d + N_DEV, N_DEV)
            for c in range(M // MCHUNK):
                rows = pl.ds(c * MCHUNK, MCHUNK)
                bounce[d, rows, :] = (
                    qrecv[d - 1, rows, :].astype(jnp.float32) * scale)
            cp = pltpu.make_async_copy(
                bounce.at[d], out_ref.at[pl.ds(src * M, M), :],
                copy_sems.at[d])
            cp.start()
            writes.append(cp)

        for cp in writes:
            cp.wait()
        for r in amax_rdmas + data_rdmas:
            r.wait_send()

    return pl.pallas_call(
        body,
        out_shape=jax.ShapeDtypeStruct((N_DEV * M, NB), jnp.float32),
        in_specs=[
            pl.BlockSpec(memory_space=pl.ANY),
            pl.BlockSpec((8, 128), memory_space=pltpu.VMEM),
        ],
        out_specs=pl.BlockSpec(memory_space=pl.ANY),
        scratch_shapes=[
            pltpu.VMEM((N_DEV, M, NB), jnp.float32),
            pltpu.VMEM((N_DEV - 1, M, NB), jnp.int8),
            pltpu.VMEM((N_DEV - 1, M, NB), jnp.int8),
            pltpu.VMEM((N_DEV, 8, 128), jnp.float32),
            pltpu.SemaphoreType.DMA((N_DEV,)),
            pltpu.SemaphoreType.DMA((N_DEV,)),
            pltpu.SemaphoreType.DMA((N_DEV,)),
            pltpu.SemaphoreType.DMA((N_DEV,)),
            pltpu.SemaphoreType.DMA((N_DEV,)),
        ],
        input_output_aliases={0: 0},
        compiler_params=pltpu.CompilerParams(
            vmem_limit_bytes=128 * 1024 * 1024,
        ),
    )(y, amax_local)


def kernel(x, w_mat):
    y, amax_local = _gemm(x, w_mat)
    return _a2a(y, amax_local)


# device time: 174952 ns/iter; 1.0250x vs baseline; 1.0218x over previous
import jax
import jax.numpy as jnp
from jax import lax
from jax.experimental import pallas as pl
from jax.experimental.pallas import tpu as pltpu

N_DEV = 4
NBLK = 512
MCHUNK = 256


def _gemm(x, w_mat):
    M, K = x.shape
    _, N = w_mat.shape

    def body(x_ref, w_ref, y_ref, amax_ref):
        n = pl.program_id(0)
        part = jnp.maximum(
            jnp.dot(x_ref[...], w_ref[...],
                    preferred_element_type=jnp.float32),
            0.0,
        )
        y_ref[...] = part
        m = jnp.full((8, 128), jnp.max(part), jnp.float32)

        @pl.when(n == 0)
        def _():
            amax_ref[...] = m

        @pl.when(n != 0)
        def _():
            amax_ref[...] = jnp.maximum(amax_ref[...], m)

    blks = (N // NBLK) // N_DEV
    return pl.pallas_call(
        body,
        grid=(N // NBLK,),
        out_shape=(
            jax.ShapeDtypeStruct((N_DEV * M, N // N_DEV), jnp.float32),
            jax.ShapeDtypeStruct((8, 128), jnp.float32),
        ),
        in_specs=[
            pl.BlockSpec((M, K), lambda n: (0, 0)),
            pl.BlockSpec((K, NBLK), lambda n: (0, n)),
        ],
        out_specs=(
            pl.BlockSpec((M, NBLK), lambda n: (n // blks, n % blks)),
            pl.BlockSpec((8, 128), lambda n: (0, 0)),
        ),
        compiler_params=pltpu.CompilerParams(
            dimension_semantics=("arbitrary",),
            vmem_limit_bytes=128 * 1024 * 1024,
        ),
    )(x, w_mat)


def _a2a(y, amax_local):
    NB = y.shape[1]
    M = y.shape[0] // N_DEV

    def body(y_ref, amax_in, out_ref, bounce, qsend, qrecv, amax_buf,
             copy_sems, dsend_sems, drecv_sems, asend_sems, arecv_sems):
        my = lax.axis_index("i")

        amax_buf[0] = amax_in[...]
        amax_rdmas = []
        for d in range(1, N_DEV):
            tgt = lax.rem(my + d, N_DEV)
            r = pltpu.make_async_remote_copy(
                src_ref=amax_buf.at[0],
                dst_ref=amax_buf.at[d],
                send_sem=asend_sems.at[d],
                recv_sem=arecv_sems.at[d],
                device_id=(tgt,),
                device_id_type=pl.DeviceIdType.MESH,
            )
            r.start()
            amax_rdmas.append(r)

        reads = [None] * N_DEV
        for d in [1, 2, 3, 0]:
            tgt = lax.rem(my + d, N_DEV)
            cp = pltpu.make_async_copy(
                y_ref.at[pl.ds(tgt * M, M), :],
                bounce.at[d],
                copy_sems.at[d],
            )
            cp.start()
            reads[d] = cp

        for r in amax_rdmas:
            r.wait_recv()
        amax = jnp.max(amax_buf[...])
        scale = amax / 127.0

        NCH = M // MCHUNK

        data_rdmas = {}
        for d in range(1, N_DEV):
            tgt = lax.rem(my + d, N_DEV)
            reads[d].wait()
            for c in range(NCH):
                rows = pl.ds(c * MCHUNK, MCHUNK)
                q = jnp.clip(jnp.round(bounce[d, rows, :] / scale),
                             -127.0, 127.0)
                qsend[d - 1, rows, :] = q.astype(jnp.int8)
                r = pltpu.make_async_remote_copy(
                    src_ref=qsend.at[d - 1, rows, :],
                    dst_ref=qrecv.at[d - 1, rows, :],
                    send_sem=dsend_sems.at[d, c],
                    recv_sem=drecv_sems.at[d, c],
                    device_id=(tgt,),
                    device_id_type=pl.DeviceIdType.MESH,
                )
                r.start()
                data_rdmas[(d, c)] = r

        reads[0].wait()
        for c in range(NCH):
            rows = pl.ds(c * MCHUNK, MCHUNK)
            q = jnp.clip(jnp.round(bounce[0, rows, :] / scale),
                         -127.0, 127.0)
            bounce[0, rows, :] = q * scale
        writes = [pltpu.make_async_copy(
            bounce.at[0], out_ref.at[pl.ds(my * M, M), :], copy_sems.at[0])]
        writes[0].start()

        for d in range(1, N_DEV):
            src = lax.rem(my - d + N_DEV, N_DEV)
            for c in range(NCH):
                rows = pl.ds(c * MCHUNK, MCHUNK)
                data_rdmas[(d, c)].wait_recv()
                bounce[d, rows, :] = (
                    qrecv[d - 1, rows, :].astype(jnp.float32) * scale)
            cp = pltpu.make_async_copy(
                bounce.at[d], out_ref.at[pl.ds(src * M, M), :],
                copy_sems.at[d])
            cp.start()
            writes.append(cp)

        for cp in writes:
            cp.wait()
        for r in amax_rdmas:
            r.wait_send()
        for r in data_rdmas.values():
            r.wait_send()

    return pl.pallas_call(
        body,
        out_shape=jax.ShapeDtypeStruct((N_DEV * M, NB), jnp.float32),
        in_specs=[
            pl.BlockSpec(memory_space=pl.ANY),
            pl.BlockSpec((8, 128), memory_space=pltpu.VMEM),
        ],
        out_specs=pl.BlockSpec(memory_space=pl.ANY),
        scratch_shapes=[
            pltpu.VMEM((N_DEV, M, NB), jnp.float32),
            pltpu.VMEM((N_DEV - 1, M, NB), jnp.int8),
            pltpu.VMEM((N_DEV - 1, M, NB), jnp.int8),
            pltpu.VMEM((N_DEV, 8, 128), jnp.float32),
            pltpu.SemaphoreType.DMA((N_DEV,)),
            pltpu.SemaphoreType.DMA((N_DEV, M // MCHUNK)),
            pltpu.SemaphoreType.DMA((N_DEV, M // MCHUNK)),
            pltpu.SemaphoreType.DMA((N_DEV,)),
            pltpu.SemaphoreType.DMA((N_DEV,)),
        ],
        input_output_aliases={0: 0},
        compiler_params=pltpu.CompilerParams(
            vmem_limit_bytes=128 * 1024 * 1024,
        ),
    )(y, amax_local)


def kernel(x, w_mat):
    y, amax_local = _gemm(x, w_mat)
    return _a2a(y, amax_local)


# device time: 170593 ns/iter; 1.0512x vs baseline; 1.0256x over previous
import jax
import jax.numpy as jnp
from jax import lax
from jax.experimental import pallas as pl
from jax.experimental.pallas import tpu as pltpu

N_DEV = 4
NBLK = 512
MCHUNK = 256


def _gemm(x, w_mat):
    M, K = x.shape
    _, N = w_mat.shape

    def body(x_ref, w_ref, y_ref, amax_ref):
        n = pl.program_id(0)
        part = jnp.maximum(
            jnp.dot(x_ref[...], w_ref[...],
                    preferred_element_type=jnp.float32),
            0.0,
        )
        y_ref[...] = part
        m = jnp.full((8, 128), jnp.max(part), jnp.float32)

        @pl.when(n == 0)
        def _():
            amax_ref[...] = m

        @pl.when(n != 0)
        def _():
            amax_ref[...] = jnp.maximum(amax_ref[...], m)

    blks = (N // NBLK) // N_DEV
    return pl.pallas_call(
        body,
        grid=(N // NBLK,),
        out_shape=(
            jax.ShapeDtypeStruct((N_DEV * M, N // N_DEV), jnp.float32),
            jax.ShapeDtypeStruct((8, 128), jnp.float32),
        ),
        in_specs=[
            pl.BlockSpec((M, K), lambda n: (0, 0)),
            pl.BlockSpec((K, NBLK), lambda n: (0, n)),
        ],
        out_specs=(
            pl.BlockSpec((M, NBLK), lambda n: (n // blks, n % blks)),
            pl.BlockSpec((8, 128), lambda n: (0, 0)),
        ),
        compiler_params=pltpu.CompilerParams(
            dimension_semantics=("arbitrary",),
            vmem_limit_bytes=128 * 1024 * 1024,
        ),
    )(x, w_mat)


def _a2a(y, amax_local):
    NB = y.shape[1]
    M = y.shape[0] // N_DEV

    def body(y_ref, amax_in, out_ref, bounce, qsend, qrecv, amax_buf,
             copy_sems, wsems, dsend_sems, drecv_sems, asend_sems,
             arecv_sems):
        my = lax.axis_index("i")

        amax_buf[0] = amax_in[...]
        amax_rdmas = []
        for d in range(1, N_DEV):
            tgt = lax.rem(my + d, N_DEV)
            r = pltpu.make_async_remote_copy(
                src_ref=amax_buf.at[0],
                dst_ref=amax_buf.at[d],
                send_sem=asend_sems.at[d],
                recv_sem=arecv_sems.at[d],
                device_id=(tgt,),
                device_id_type=pl.DeviceIdType.MESH,
            )
            r.start()
            amax_rdmas.append(r)

        NCH = M // MCHUNK

        reads = {}
        for d in [1, 2, 3, 0]:
            tgt = lax.rem(my + d, N_DEV)
            for c in range(NCH):
                rows = pl.ds(c * MCHUNK, MCHUNK)
                cp = pltpu.make_async_copy(
                    y_ref.at[pl.ds(tgt * M + c * MCHUNK, MCHUNK), :],
                    bounce.at[d, rows, :],
                    copy_sems.at[d, c],
                )
                cp.start()
                reads[(d, c)] = cp

        for r in amax_rdmas:
            r.wait_recv()
        amax = jnp.max(amax_buf[...])
        scale = amax / 127.0

        data_rdmas = {}
        for d in range(1, N_DEV):
            tgt = lax.rem(my + d, N_DEV)
            for c in range(NCH):
                rows = pl.ds(c * MCHUNK, MCHUNK)
                reads[(d, c)].wait()
                q = jnp.clip(jnp.round(bounce[d, rows, :] / scale),
                             -127.0, 127.0)
                qsend[d - 1, rows, :] = q.astype(jnp.int8)
                r = pltpu.make_async_remote_copy(
                    src_ref=qsend.at[d - 1, rows, :],
                    dst_ref=qrecv.at[d - 1, rows, :],
                    send_sem=dsend_sems.at[d, c],
                    recv_sem=drecv_sems.at[d, c],
                    device_id=(tgt,),
                    device_id_type=pl.DeviceIdType.MESH,
                )
                r.start()
                data_rdmas[(d, c)] = r

        writes = []
        for c in range(NCH):
            rows = pl.ds(c * MCHUNK, MCHUNK)
            reads[(0, c)].wait()
            q = jnp.clip(jnp.round(bounce[0, rows, :] / scale),
                         -127.0, 127.0)
            bounce[0, rows, :] = q * scale
            cp = pltpu.make_async_copy(
                bounce.at[0, rows, :],
                out_ref.at[pl.ds(my * M + c * MCHUNK, MCHUNK), :],
                wsems.at[0, c])
            cp.start()
            writes.append(cp)

        for d in range(1, N_DEV):
            src = lax.rem(my - d + N_DEV, N_DEV)
            for c in range(NCH):
                rows = pl.ds(c * MCHUNK, MCHUNK)
                data_rdmas[(d, c)].wait_recv()
                bounce[d, rows, :] = (
                    qrecv[d - 1, rows, :].astype(jnp.float32) * scale)
                cp = pltpu.make_async_copy(
                    bounce.at[d, rows, :],
                    out_ref.at[pl.ds(src * M + c * MCHUNK, MCHUNK), :],
                    wsems.at[d, c])
                cp.start()
                writes.append(cp)

        for cp in writes:
            cp.wait()
        for r in amax_rdmas:
            r.wait_send()
        for r in data_rdmas.values():
            r.wait_send()

    return pl.pallas_call(
        body,
        out_shape=jax.ShapeDtypeStruct((N_DEV * M, NB), jnp.float32),
        in_specs=[
            pl.BlockSpec(memory_space=pl.ANY),
            pl.BlockSpec((8, 128), memory_space=pltpu.VMEM),
        ],
        out_specs=pl.BlockSpec(memory_space=pl.ANY),
        scratch_shapes=[
            pltpu.VMEM((N_DEV, M, NB), jnp.float32),
            pltpu.VMEM((N_DEV - 1, M, NB), jnp.int8),
            pltpu.VMEM((N_DEV - 1, M, NB), jnp.int8),
            pltpu.VMEM((N_DEV, 8, 128), jnp.float32),
            pltpu.SemaphoreType.DMA((N_DEV, M // MCHUNK)),
            pltpu.SemaphoreType.DMA((N_DEV, M // MCHUNK)),
            pltpu.SemaphoreType.DMA((N_DEV, M // MCHUNK)),
            pltpu.SemaphoreType.DMA((N_DEV, M // MCHUNK)),
            pltpu.SemaphoreType.DMA((N_DEV,)),
            pltpu.SemaphoreType.DMA((N_DEV,)),
        ],
        input_output_aliases={0: 0},
        compiler_params=pltpu.CompilerParams(
            vmem_limit_bytes=128 * 1024 * 1024,
        ),
    )(y, amax_local)


def kernel(x, w_mat):
    y, amax_local = _gemm(x, w_mat)
    return _a2a(y, amax_local)
